# baseline (speedup 1.0000x reference)
import functools

import jax
import jax.numpy as jnp
import numpy as np
from jax import lax
from jax.experimental import pallas as pl
from jax.experimental.pallas import tpu as pltpu
from jax.experimental.pallas import tpu_sc as plsc

N_NODES = 100000
MAX_DEGREE = 32
BATCH = 16384
NUM_SAMPLES = 16

NC, NS = 2, 16
NW = NC * NS
B_HALF = BATCH // 2

_PERM16 = np.array([31, 7, 4, 29, 16, 19, 2, 5, 30, 3, 22, 6, 18, 10, 11, 15],
                   dtype=np.int32)

_mesh = plsc.VectorSubcoreMesh(core_axis_name="c", subcore_axis_name="s")


@functools.partial(
    pl.kernel,
    out_type=jax.ShapeDtypeStruct((NUM_SAMPLES, BATCH), jnp.int32),
    mesh=_mesh,
    scratch_types=[
        pltpu.VMEM((B_HALF,), jnp.int32),
        pltpu.VMEM((N_NODES,), jnp.int32),
        pltpu.VMEM((B_HALF,), jnp.int32),
    ],
    compiler_params=pltpu.CompilerParams(
        needs_layout_passes=False, use_tc_tiling_on_sc=True),
)
def _sample_neighbors(adjt_hbm, ids_hbm, out_hbm, ids_v, col_v, out_v):
    wid = lax.axis_index("s") * NC + lax.axis_index("c")
    j = wid >> 1
    h = wid & 1

    pltpu.sync_copy(ids_hbm.at[h], ids_v)
    pltpu.sync_copy(adjt_hbm.at[j], col_v)

    def gather_block(r, carry):
        o = r * np.int32(16)
        idx = ids_v[pl.ds(o, 16)]
        out_v[pl.ds(o, 16)] = plsc.load_gather(col_v, [idx])
        return carry

    lax.fori_loop(jnp.int32(0), jnp.int32(B_HALF // 16), gather_block,
                  jnp.int32(0))

    pltpu.sync_copy(out_v, out_hbm.at[j, pl.ds(h * np.int32(B_HALF), B_HALF)])


def kernel(ids, num_samples, adj_info):
    del num_samples
    ids32 = ids.astype(jnp.int32).reshape(2, B_HALF)
    adjt32 = adj_info.T[jnp.asarray(_PERM16)].astype(jnp.int32)
    outt = _sample_neighbors(adjt32, ids32)
    return outt.T.astype(jnp.int64)


# speedup vs baseline: 1.0677x; 1.0677x over previous
import functools

import jax
import jax.numpy as jnp
import numpy as np
from jax import lax
from jax.experimental import pallas as pl
from jax.experimental.pallas import tpu as pltpu
from jax.experimental.pallas import tpu_sc as plsc

N_NODES = 100000
MAX_DEGREE = 32
BATCH = 16384
NUM_SAMPLES = 16

NC, NS = 2, 16
NW = NC * NS
B_HALF = BATCH // 2

_PERM16 = np.array([31, 7, 4, 29, 16, 19, 2, 5, 30, 3, 22, 6, 18, 10, 11, 15],
                   dtype=np.int32)

_mesh = plsc.VectorSubcoreMesh(core_axis_name="c", subcore_axis_name="s")


@functools.partial(
    pl.kernel,
    out_type=jax.ShapeDtypeStruct((NUM_SAMPLES, BATCH), jnp.int32),
    mesh=_mesh,
    scratch_types=[
        pltpu.VMEM((B_HALF,), jnp.int32),
        pltpu.VMEM((N_NODES,), jnp.int32),
        pltpu.VMEM((B_HALF,), jnp.int32),
        pltpu.VMEM((NUM_SAMPLES,), jnp.int32),
    ],
    compiler_params=pltpu.CompilerParams(
        needs_layout_passes=False, use_tc_tiling_on_sc=True),
)
def _sample_neighbors(adjt_hbm, ids_hbm, perm_hbm, out_hbm,
                      ids_v, col_v, out_v, perm_v):
    wid = lax.axis_index("s") * NC + lax.axis_index("c")
    j = wid >> 1
    h = wid & 1

    pltpu.sync_copy(perm_hbm, perm_v)
    pj = jnp.max(plsc.load_gather(perm_v, [jnp.full((16,), j, jnp.int32)]))

    pltpu.sync_copy(ids_hbm.at[h], ids_v)
    pltpu.sync_copy(adjt_hbm.at[pj], col_v)

    @plsc.parallel_loop(jnp.int32(0), jnp.int32(B_HALF), step=jnp.int32(16), unroll=4)
    def gather_block(o):
        idx = ids_v[pl.ds(o, 16)]
        out_v[pl.ds(o, 16)] = plsc.load_gather(col_v, [idx])

    pltpu.sync_copy(out_v, out_hbm.at[j, pl.ds(h * np.int32(B_HALF), B_HALF)])


def kernel(ids, num_samples, adj_info):
    del num_samples
    ids32 = ids.astype(jnp.int32).reshape(2, B_HALF)
    adjt32 = lax.bitcast_convert_type(adj_info.T.astype(jnp.uint32), jnp.int32)
    outt = _sample_neighbors(adjt32, ids32, jnp.asarray(_PERM16))
    return outt.T.astype(jnp.int64)
